# baseline (speedup 1.0000x reference)
import numpy as np
import jax
import jax.numpy as jnp
from jax.experimental import pallas as pl
from jax.experimental.pallas import tpu as pltpu

_N = 10000
_NH1 = 64
_NH2 = 32
_BR = 400
_CC = 1024
_JR = _N // _BR
_KC = -(-_N // _CC)
_NP = _KC * _CC


def _make_schedule():
    rows = []
    for j in range(_JR):
        for k in range(_KC):
            rows.append((j, k, 0))
    for j in range(_JR):
        fused = (j * _BR) // _CC
        for k in range(fused, _KC):
            rows.append((j, k, 1))
    return np.asarray(rows, dtype=np.int32)

_SCHED = _make_schedule()


def _s1_kernel(feature_ref, w1_ref, s1_ref):
    f = feature_ref[...].astype(jnp.bfloat16)
    w = w1_ref[...].astype(jnp.bfloat16)
    s1_ref[0:_N, :] = jnp.dot(f, w, preferred_element_type=jnp.float32).astype(
        jnp.bfloat16
    )
    s1_ref[_N:_NP, :] = jnp.zeros((_NP - _N, _NH1), jnp.bfloat16)


def _fused_kernel(sched_ref, adj_ref, s1_ref, b1_ref, w2_ref, b2_ref,
                  out_ref, xacc_ref, s2_ref):
    t = pl.program_id(0)
    j = sched_ref[t, 0]
    k = sched_ref[t, 1]
    ph = sched_ref[t, 2]
    col0 = k * _CC

    cols = col0 + jax.lax.broadcasted_iota(jnp.int32, (_BR, _CC), 1)
    a = jnp.where(cols < _N, adj_ref[...], 0.0).astype(jnp.bfloat16)

    @pl.when(t == 0)
    def _init():
        out_ref[...] = jnp.broadcast_to(b2_ref[...], out_ref.shape)
        s2_ref[_N:_NP, :] = jnp.zeros((_NP - _N, _NH2), jnp.bfloat16)

    @pl.when(ph == 0)
    def _pass1():
        s1_blk = s1_ref[pl.ds(col0, _CC), :]
        part = jnp.dot(a, s1_blk, preferred_element_type=jnp.float32)

        @pl.when(k == 0)
        def _():
            xacc_ref[...] = part

        @pl.when(k > 0)
        def _():
            xacc_ref[...] += part

        @pl.when(k == _KC - 1)
        def _():
            x = jnp.maximum(xacc_ref[...] + b1_ref[...], 0.0)
            s2_ref[pl.ds(j * _BR, _BR), :] = jnp.dot(
                x.astype(jnp.bfloat16), w2_ref[...],
                preferred_element_type=jnp.float32,
            ).astype(jnp.bfloat16)

    @pl.when((ph == 1) | ((k + 1) * _CC <= j * _BR))
    def _pass2():
        s2_blk = s2_ref[pl.ds(col0, _CC), :]
        out_ref[pl.ds(j * _BR, _BR), :] += jnp.dot(
            a, s2_blk, preferred_element_type=jnp.float32
        )


@jax.jit
def kernel(feature, adj, W1, b1, W2, b2):
    n, nfeat = feature.shape
    b1r = b1.reshape(1, _NH1)
    b2r = b2.reshape(1, _NH2)

    s1 = pl.pallas_call(
        _s1_kernel,
        out_shape=jax.ShapeDtypeStruct((_NP, _NH1), jnp.bfloat16),
    )(feature, W1)

    sched = jnp.asarray(_SCHED)
    grid_spec = pltpu.PrefetchScalarGridSpec(
        num_scalar_prefetch=1,
        grid=(_SCHED.shape[0],),
        in_specs=[
            pl.BlockSpec((_BR, _CC), lambda t, s: (s[t, 0], s[t, 1])),
            pl.BlockSpec((_NP, _NH1), lambda t, s: (0, 0)),
            pl.BlockSpec((1, _NH1), lambda t, s: (0, 0)),
            pl.BlockSpec((_NH1, _NH2), lambda t, s: (0, 0)),
            pl.BlockSpec((1, _NH2), lambda t, s: (0, 0)),
        ],
        out_specs=pl.BlockSpec((_N, _NH2), lambda t, s: (0, 0)),
        scratch_shapes=[
            pltpu.VMEM((_BR, _NH1), jnp.float32),
            pltpu.VMEM((_NP, _NH2), jnp.bfloat16),
        ],
    )
    out = pl.pallas_call(
        _fused_kernel,
        grid_spec=grid_spec,
        out_shape=jax.ShapeDtypeStruct((n, _NH2), jnp.float32),
        compiler_params=pltpu.CompilerParams(
            dimension_semantics=("arbitrary",),
        ),
    )(sched, adj, s1, b1r, W2.astype(jnp.bfloat16), b2r)
    return out


# speedup vs baseline: 1.7220x; 1.7220x over previous
import numpy as np
import jax
import jax.numpy as jnp
from jax.experimental import pallas as pl
from jax.experimental.pallas import tpu as pltpu

_N = 10000
_NH1 = 64
_NH2 = 32
_BR = 2000
_CC = 1920
_JR = _N // _BR
_KC = -(-_N // _CC)
_NP = _KC * _CC


def _make_schedule():
    rows = []
    for j in range(_JR):
        for k in range(_KC):
            rows.append((j, k, 0))
    for j in range(_JR):
        fused = (j * _BR) // _CC
        for k in range(fused, _KC):
            rows.append((j, k, 1))
    return np.asarray(rows, dtype=np.int32)

_SCHED = _make_schedule()


def _s1_kernel(feature_ref, w1_ref, s1_ref):
    f = feature_ref[...].astype(jnp.bfloat16)
    w = w1_ref[...].astype(jnp.bfloat16)
    s1_ref[0:_N, :] = jnp.dot(f, w, preferred_element_type=jnp.float32).astype(
        jnp.bfloat16
    )
    s1_ref[_N:_NP, :] = jnp.zeros((_NP - _N, _NH1), jnp.bfloat16)


def _fused_kernel(sched_ref, adj_ref, s1_ref, b1_ref, w2_ref, b2_ref,
                  out_ref, xacc_ref, s2_ref):
    t = pl.program_id(0)
    j = sched_ref[t, 0]
    k = sched_ref[t, 1]
    ph = sched_ref[t, 2]
    col0 = k * _CC

    cols = col0 + jax.lax.broadcasted_iota(jnp.int32, (_BR, _CC), 1)
    a = jnp.where(cols < _N, adj_ref[...], 0.0).astype(jnp.bfloat16)

    @pl.when(t == 0)
    def _init():
        out_ref[...] = jnp.broadcast_to(b2_ref[...], out_ref.shape)
        s2_ref[_N:_NP, :] = jnp.zeros((_NP - _N, _NH2), jnp.bfloat16)

    @pl.when(ph == 0)
    def _pass1():
        s1_blk = s1_ref[pl.ds(col0, _CC), :]
        part = jnp.dot(a, s1_blk, preferred_element_type=jnp.float32)

        @pl.when(k == 0)
        def _():
            xacc_ref[...] = part

        @pl.when(k > 0)
        def _():
            xacc_ref[...] += part

        @pl.when(k == _KC - 1)
        def _():
            x = jnp.maximum(xacc_ref[...] + b1_ref[...], 0.0)
            s2_ref[pl.ds(j * _BR, _BR), :] = jnp.dot(
                x.astype(jnp.bfloat16), w2_ref[...],
                preferred_element_type=jnp.float32,
            ).astype(jnp.bfloat16)

    @pl.when((ph == 1) | ((k + 1) * _CC <= j * _BR))
    def _pass2():
        s2_blk = s2_ref[pl.ds(col0, _CC), :]
        out_ref[pl.ds(j * _BR, _BR), :] += jnp.dot(
            a, s2_blk, preferred_element_type=jnp.float32
        )


@jax.jit
def kernel(feature, adj, W1, b1, W2, b2):
    n, nfeat = feature.shape
    b1r = b1.reshape(1, _NH1)
    b2r = b2.reshape(1, _NH2)

    s1 = pl.pallas_call(
        _s1_kernel,
        out_shape=jax.ShapeDtypeStruct((_NP, _NH1), jnp.bfloat16),
    )(feature, W1)

    sched = jnp.asarray(_SCHED)
    grid_spec = pltpu.PrefetchScalarGridSpec(
        num_scalar_prefetch=1,
        grid=(_SCHED.shape[0],),
        in_specs=[
            pl.BlockSpec((_BR, _CC), lambda t, s: (s[t, 0], s[t, 1])),
            pl.BlockSpec((_NP, _NH1), lambda t, s: (0, 0)),
            pl.BlockSpec((1, _NH1), lambda t, s: (0, 0)),
            pl.BlockSpec((_NH1, _NH2), lambda t, s: (0, 0)),
            pl.BlockSpec((1, _NH2), lambda t, s: (0, 0)),
        ],
        out_specs=pl.BlockSpec((_N, _NH2), lambda t, s: (0, 0)),
        scratch_shapes=[
            pltpu.VMEM((_BR, _NH1), jnp.float32),
            pltpu.VMEM((_NP, _NH2), jnp.bfloat16),
        ],
    )
    out = pl.pallas_call(
        _fused_kernel,
        grid_spec=grid_spec,
        out_shape=jax.ShapeDtypeStruct((n, _NH2), jnp.float32),
        compiler_params=pltpu.CompilerParams(
            dimension_semantics=("arbitrary",),
        ),
    )(sched, adj, s1, b1r, W2.astype(jnp.bfloat16), b2r)
    return out


# speedup vs baseline: 1.8312x; 1.0634x over previous
import numpy as np
import jax
import jax.numpy as jnp
from jax.experimental import pallas as pl
from jax.experimental.pallas import tpu as pltpu

_N = 10000
_NH1 = 64
_NH2 = 32
_BR = 2000
_CC = 1920
_JR = _N // _BR
_KC = -(-_N // _CC)
_NP = _KC * _CC


def _make_schedule():
    rows = []
    for j in range(_JR):
        for k in range(_KC):
            rows.append((j, k, 0))
    for j in range(_JR):
        fused = (j * _BR) // _CC
        for k in range(fused, _KC):
            rows.append((j, k, 1))
    return np.asarray(rows, dtype=np.int32)

_SCHED = _make_schedule()


def _s1_kernel(feature_ref, w1_ref, s1_ref):
    f = feature_ref[...].astype(jnp.bfloat16)
    w = w1_ref[...].astype(jnp.bfloat16)
    s1_ref[0:_N, :] = jnp.dot(f, w, preferred_element_type=jnp.float32).astype(
        jnp.bfloat16
    )
    s1_ref[_N:_NP, :] = jnp.zeros((_NP - _N, _NH1), jnp.bfloat16)


def _fused_kernel(sched_ref, adj_ref, s1_ref, b1_ref, w2_ref, b2_ref,
                  out_ref, xacc_ref, ss_ref):
    t = pl.program_id(0)
    j = sched_ref[t, 0]
    k = sched_ref[t, 1]
    ph = sched_ref[t, 2]
    col0 = k * _CC

    @pl.when(t == 0)
    def _init():
        ss_ref[:, 0:_NH2] = jnp.zeros((_NP, _NH2), jnp.bfloat16)
        ss_ref[:, _NH2:_NH2 + _NH1] = s1_ref[...]
        ss_ref[:, _NH2 + _NH1:] = jnp.zeros((_NP, 128 - _NH1 - _NH2),
                                            jnp.bfloat16)
        out_ref[...] = jnp.broadcast_to(b2_ref[...], out_ref.shape)

    a = adj_ref[...].astype(jnp.bfloat16)
    res = jnp.dot(a, ss_ref[pl.ds(col0, _CC), :],
                  preferred_element_type=jnp.float32)

    @pl.when(ph == 0)
    def _pass1():
        @pl.when(k == 0)
        def _():
            xacc_ref[...] = res

        @pl.when(k > 0)
        def _():
            xacc_ref[...] += res

        @pl.when(k == _KC - 1)
        def _():
            x = jnp.maximum(
                xacc_ref[:, _NH2:_NH2 + _NH1] + b1_ref[...], 0.0)
            ss_ref[pl.ds(j * _BR, _BR), 0:_NH2] = jnp.dot(
                x.astype(jnp.bfloat16), w2_ref[...],
                preferred_element_type=jnp.float32,
            ).astype(jnp.bfloat16)

    @pl.when((ph == 1) | ((k + 1) * _CC <= j * _BR))
    def _pass2():
        out_ref[pl.ds(j * _BR, _BR), :] += res[:, 0:_NH2]


@jax.jit
def kernel(feature, adj, W1, b1, W2, b2):
    n, nfeat = feature.shape
    b1r = b1.reshape(1, _NH1)
    b2r = b2.reshape(1, _NH2)

    s1 = pl.pallas_call(
        _s1_kernel,
        out_shape=jax.ShapeDtypeStruct((_NP, _NH1), jnp.bfloat16),
    )(feature, W1)

    sched = jnp.asarray(_SCHED)
    grid_spec = pltpu.PrefetchScalarGridSpec(
        num_scalar_prefetch=1,
        grid=(_SCHED.shape[0],),
        in_specs=[
            pl.BlockSpec((_BR, _CC), lambda t, s: (s[t, 0], s[t, 1])),
            pl.BlockSpec((_NP, _NH1), lambda t, s: (0, 0)),
            pl.BlockSpec((1, _NH1), lambda t, s: (0, 0)),
            pl.BlockSpec((_NH1, _NH2), lambda t, s: (0, 0)),
            pl.BlockSpec((1, _NH2), lambda t, s: (0, 0)),
        ],
        out_specs=pl.BlockSpec((n, _NH2), lambda t, s: (0, 0)),
        scratch_shapes=[
            pltpu.VMEM((_BR, 128), jnp.float32),
            pltpu.VMEM((_NP, 128), jnp.bfloat16),
        ],
    )
    out = pl.pallas_call(
        _fused_kernel,
        grid_spec=grid_spec,
        out_shape=jax.ShapeDtypeStruct((n, _NH2), jnp.float32),
        compiler_params=pltpu.CompilerParams(
            dimension_semantics=("arbitrary",),
        ),
    )(sched, adj, s1, b1r, W2.astype(jnp.bfloat16), b2r)
    return out


# speedup vs baseline: 1.8774x; 1.0252x over previous
import numpy as np
import jax
import jax.numpy as jnp
from jax.experimental import pallas as pl
from jax.experimental.pallas import tpu as pltpu

_N = 10000
_NH1 = 64
_NH2 = 32
_BR = 2000
_CC = 1920
_JR = _N // _BR
_KC = -(-_N // _CC)
_NP = _KC * _CC


def _make_schedule():
    rows = []
    for j in range(_JR):
        for k in range(_KC):
            rows.append((j, k, 0))
    for j in range(_JR):
        fused = (j * _BR) // _CC
        for k in range(fused, _KC):
            rows.append((j, k, 1))
    return np.asarray(rows, dtype=np.int32)

_SCHED = _make_schedule()


def _fused_kernel(sched_ref, adj_ref, feat_ref, w1_ref, b1_ref, w2_ref,
                  b2_ref, out_ref, xacc_ref, ss_ref):
    t = pl.program_id(0)
    j = sched_ref[t, 0]
    k = sched_ref[t, 1]
    ph = sched_ref[t, 2]
    col0 = k * _CC

    @pl.when(t == 0)
    def _init():
        ss_ref[...] = jnp.zeros((_NP, 128), jnp.bfloat16)
        s1v = jnp.dot(feat_ref[...], w1_ref[...],
                      preferred_element_type=jnp.float32)
        ss_ref[0:_N, _NH2:_NH2 + _NH1] = s1v.astype(jnp.bfloat16)
        out_ref[...] = jnp.broadcast_to(b2_ref[...], out_ref.shape)

    a = adj_ref[...].astype(jnp.bfloat16)
    res = jnp.dot(a, ss_ref[pl.ds(col0, _CC), :],
                  preferred_element_type=jnp.float32)

    @pl.when(ph == 0)
    def _pass1():
        @pl.when(k == 0)
        def _():
            xacc_ref[...] = res

        @pl.when(k > 0)
        def _():
            xacc_ref[...] += res

        @pl.when(k == _KC - 1)
        def _():
            x = jnp.maximum(
                xacc_ref[:, _NH2:_NH2 + _NH1] + b1_ref[...], 0.0)
            ss_ref[pl.ds(j * _BR, _BR), 0:_NH2] = jnp.dot(
                x.astype(jnp.bfloat16), w2_ref[...],
                preferred_element_type=jnp.float32,
            ).astype(jnp.bfloat16)

    @pl.when((ph == 1) | ((k + 1) * _CC <= j * _BR))
    def _pass2():
        out_ref[pl.ds(j * _BR, _BR), :] += res[:, 0:_NH2]


@jax.jit
def kernel(feature, adj, W1, b1, W2, b2):
    n, nfeat = feature.shape
    b1r = b1.reshape(1, _NH1)
    b2r = b2.reshape(1, _NH2)

    sched = jnp.asarray(_SCHED)
    grid_spec = pltpu.PrefetchScalarGridSpec(
        num_scalar_prefetch=1,
        grid=(_SCHED.shape[0],),
        in_specs=[
            pl.BlockSpec((_BR, _CC), lambda t, s: (s[t, 0], s[t, 1])),
            pl.BlockSpec((n, nfeat), lambda t, s: (0, 0)),
            pl.BlockSpec((nfeat, _NH1), lambda t, s: (0, 0)),
            pl.BlockSpec((1, _NH1), lambda t, s: (0, 0)),
            pl.BlockSpec((_NH1, _NH2), lambda t, s: (0, 0)),
            pl.BlockSpec((1, _NH2), lambda t, s: (0, 0)),
        ],
        out_specs=pl.BlockSpec((n, _NH2), lambda t, s: (0, 0)),
        scratch_shapes=[
            pltpu.VMEM((_BR, 128), jnp.float32),
            pltpu.VMEM((_NP, 128), jnp.bfloat16),
        ],
    )
    out = pl.pallas_call(
        _fused_kernel,
        grid_spec=grid_spec,
        out_shape=jax.ShapeDtypeStruct((n, _NH2), jnp.float32),
        compiler_params=pltpu.CompilerParams(
            dimension_semantics=("arbitrary",),
        ),
    )(sched, adj, feature.astype(jnp.bfloat16), W1.astype(jnp.bfloat16),
      b1r, W2.astype(jnp.bfloat16), b2r)
    return out
